# Initial kernel scaffold; baseline (speedup 1.0000x reference)
import jax
import jax.numpy as jnp
from jax.experimental import pallas as pl


def kernel(points, features, knn_indices, W, b):
    raise NotImplementedError("write your pallas kernel here")


# speedup vs baseline: 23.5988x; 23.5988x over previous
import functools

import jax
import jax.numpy as jnp
from jax import lax
from jax.experimental import pallas as pl
from jax.experimental.pallas import tpu as pltpu
from jax.experimental.pallas import tpu_sc as plsc

NPTS = 20000
KNN = 16
CHUNK_PTS = 8
CHUNK_EDGES = CHUNK_PTS * KNN
NCHUNKS = NPTS // CHUNK_PTS
NWORKERS = 32
ITERS = -(-NCHUNKS // NWORKERS)
PPAD = NPTS + 16

_OUT_D = 128


def _build_sc_call():
    mesh = plsc.VectorSubcoreMesh(core_axis_name="c", subcore_axis_name="s")

    @functools.partial(
        pl.kernel,
        mesh=mesh,
        out_type=jax.ShapeDtypeStruct((NPTS, _OUT_D), jnp.float32),
        compiler_params=pltpu.CompilerParams(use_tc_tiling_on_sc=False),
        scratch_types=[
            pltpu.VMEM((CHUNK_EDGES,), jnp.int32),
            pltpu.VMEM((CHUNK_EDGES, 64), jnp.float32),
            pltpu.VMEM((CHUNK_EDGES,), jnp.float32),
            pltpu.VMEM((CHUNK_EDGES,), jnp.float32),
            pltpu.VMEM((CHUNK_EDGES,), jnp.float32),
            pltpu.VMEM((16,), jnp.float32),
            pltpu.VMEM((16,), jnp.float32),
            pltpu.VMEM((16,), jnp.float32),
            pltpu.VMEM((4, 64), jnp.float32),
            pltpu.VMEM((64,), jnp.float32),
            pltpu.VMEM((CHUNK_PTS, _OUT_D), jnp.float32),
            pltpu.SemaphoreType.DMA,
            pltpu.SemaphoreType.DMA,
        ],
    )
    def lfa_kernel(idx_hbm, feat_hbm, px_hbm, py_hbm, pz_hbm, wt_hbm, b_hbm,
                   out_hbm,
                   idx_v, featbuf, nbx, nby, nbz, ctx_v, cty_v, ctz_v,
                   wtbuf, bbuf, outbuf, sem_f, sem_p):
        wid = lax.axis_index("s") * 2 + lax.axis_index("c")
        pltpu.sync_copy(wt_hbm, wtbuf)
        pltpu.sync_copy(b_hbm, bbuf)
        wvec = [[wtbuf[ci, pl.ds(16 * v, 16)] for ci in range(4)]
                for v in range(4)]
        bvec = [bbuf[pl.ds(16 * v, 16)] for v in range(4)]

        def chunk_body(i, carry):
            c = wid + NWORKERS * i

            @pl.when(c < NCHUNKS)
            def _():
                pltpu.sync_copy(idx_hbm.at[pl.ds(c * CHUNK_EDGES, CHUNK_EDGES)],
                                idx_v)
                cp_f = pltpu.async_copy(feat_hbm.at[idx_v], featbuf, sem_f)
                cp_x = pltpu.async_copy(px_hbm.at[idx_v], nbx, sem_p)
                cp_y = pltpu.async_copy(py_hbm.at[idx_v], nby, sem_p)
                cp_z = pltpu.async_copy(pz_hbm.at[idx_v], nbz, sem_p)
                pltpu.sync_copy(px_hbm.at[pl.ds(c * CHUNK_PTS, 16)], ctx_v)
                pltpu.sync_copy(py_hbm.at[pl.ds(c * CHUNK_PTS, 16)], cty_v)
                pltpu.sync_copy(pz_hbm.at[pl.ds(c * CHUNK_PTS, 16)], ctz_v)
                cp_x.wait()
                cp_y.wait()
                cp_z.wait()
                cp_f.wait()
                cxv = ctx_v[...]
                cyv = cty_v[...]
                czv = ctz_v[...]

                for p in range(CHUNK_PTS):
                    nx = nbx[pl.ds(p * KNN, KNN)]
                    ny = nby[pl.ds(p * KNN, KNN)]
                    nz = nbz[pl.ds(p * KNN, KNN)]
                    dx = cxv[p] - nx
                    dy = cyv[p] - ny
                    dz = czv[p] - nz
                    s = dx * dx + dy * dy + dz * dz
                    bits = lax.bitcast_convert_type(s, jnp.int32)
                    seed = jnp.int32(0x5F3759DF) - (bits >> 1)
                    r = lax.bitcast_convert_type(seed, jnp.float32)
                    hs = s * jnp.float32(-0.5)
                    r = r * (hs * r * r + jnp.float32(1.5))
                    r = r * (hs * r * r + jnp.float32(1.5))
                    r = r * (hs * r * r + jnp.float32(1.5))
                    nr = s * r
                    acc_s = [jnp.zeros((16,), jnp.float32) for _ in range(4)]
                    acc_a = [jnp.zeros((16,), jnp.float32) for _ in range(4)]
                    acc_f = [jnp.zeros((16,), jnp.float32) for _ in range(4)]
                    base = p * KNN
                    for k in range(KNN):
                        dxk = dx[k]
                        dyk = dy[k]
                        dzk = dz[k]
                        nrk = nr[k]
                        for v in range(4):
                            z = (dxk * wvec[v][0] + dyk * wvec[v][1]
                                 + dzk * wvec[v][2] + nrk * wvec[v][3]
                                 + bvec[v])
                            acc_s[v] = acc_s[v] + z
                            acc_a[v] = acc_a[v] + jnp.abs(z)
                            acc_f[v] = acc_f[v] + featbuf[base + k,
                                                          pl.ds(16 * v, 16)]
                    for v in range(4):
                        outbuf[p, pl.ds(16 * v, 16)] = (
                            acc_s[v] * jnp.float32(0.6 / KNN)
                            + acc_a[v] * jnp.float32(0.4 / KNN))
                        outbuf[p, pl.ds(64 + 16 * v, 16)] = (
                            acc_f[v] * jnp.float32(1.0 / KNN))
                pltpu.sync_copy(outbuf,
                                out_hbm.at[pl.ds(c * CHUNK_PTS, CHUNK_PTS)])
            return carry

        lax.fori_loop(0, ITERS, chunk_body, 0)

    return lfa_kernel


_SC_CALL = _build_sc_call()


def kernel(points, features, knn_indices, W, b):
    B, N, D = points.shape
    pts = points.reshape(B * N, D).astype(jnp.float32)
    feat_flat = features.reshape(B * N, features.shape[-1]).astype(jnp.float32)
    pad = jnp.zeros((PPAD - B * N,), jnp.float32)
    px = jnp.concatenate([pts[:, 0], pad])
    py = jnp.concatenate([pts[:, 1], pad])
    pz = jnp.concatenate([pts[:, 2], pad])
    offs = (jnp.arange(B, dtype=jnp.int32) * N)[:, None, None]
    idx_flat = (knn_indices.astype(jnp.int32) + offs).reshape(-1)
    wt = W.astype(jnp.float32).T
    out = _SC_CALL(idx_flat, feat_flat, px, py, pz, wt,
                   b.astype(jnp.float32))
    return out.reshape(B, N, _OUT_D)


# speedup vs baseline: 29.5488x; 1.2521x over previous
import functools

import jax
import jax.numpy as jnp
from jax import lax
from jax.experimental import pallas as pl
from jax.experimental.pallas import tpu as pltpu
from jax.experimental.pallas import tpu_sc as plsc

NPTS = 20000
KNN = 16
CHUNK_PTS = 8
CHUNK_EDGES = CHUNK_PTS * KNN
NCHUNKS = NPTS // CHUNK_PTS
NWORKERS = 32
VITERS = 2 * (-(-NCHUNKS // NWORKERS) // 2 + 1)
META_W = CHUNK_EDGES + 48

_OUT_D = 128


def _build_sc_call():
    mesh = plsc.VectorSubcoreMesh(core_axis_name="c", subcore_axis_name="s")

    @functools.partial(
        pl.kernel,
        mesh=mesh,
        out_type=jax.ShapeDtypeStruct((NPTS, _OUT_D), jnp.float32),
        compiler_params=pltpu.CompilerParams(use_tc_tiling_on_sc=False),
        scratch_types=[
            pltpu.VMEM((2, META_W), jnp.int32),
            pltpu.VMEM((2, CHUNK_EDGES, 64), jnp.float32),
            pltpu.VMEM((2, CHUNK_EDGES), jnp.float32),
            pltpu.VMEM((2, CHUNK_EDGES), jnp.float32),
            pltpu.VMEM((2, CHUNK_EDGES), jnp.float32),
            pltpu.VMEM((4, 64), jnp.float32),
            pltpu.VMEM((64,), jnp.float32),
            pltpu.VMEM((2, CHUNK_PTS, _OUT_D), jnp.float32),
            pltpu.SemaphoreType.DMA,
            pltpu.SemaphoreType.DMA,
            pltpu.SemaphoreType.DMA,
            pltpu.SemaphoreType.DMA,
            pltpu.SemaphoreType.DMA,
            pltpu.SemaphoreType.DMA,
        ],
    )
    def lfa_kernel(meta_hbm, feat_hbm, px_hbm, py_hbm, pz_hbm, wt_hbm, b_hbm,
                   out_hbm,
                   meta_v, featbuf, nbx, nby, nbz, wtbuf, bbuf, outbuf,
                   sem_m0, sem_m1, sem_g0, sem_g1, sem_o0, sem_o1):
        sem_m = (sem_m0, sem_m1)
        sem_g = (sem_g0, sem_g1)
        sem_o = (sem_o0, sem_o1)
        wid = lax.axis_index("s") * 2 + lax.axis_index("c")
        pltpu.sync_copy(wt_hbm, wtbuf)
        pltpu.sync_copy(b_hbm, bbuf)
        wvec = [[wtbuf[ci, pl.ds(16 * v, 16)] for ci in range(4)]
                for v in range(4)]
        bvec = [bbuf[pl.ds(16 * v, 16)] for v in range(4)]

        def meta_copy(b, c):
            return pltpu.make_async_copy(
                meta_hbm.at[pl.ds(c * META_W, META_W)],
                meta_v.at[b], sem_m[b])

        def gather_copies(b):
            idx_ref = meta_v.at[b, pl.ds(0, CHUNK_EDGES)]
            return (
                pltpu.make_async_copy(feat_hbm.at[idx_ref], featbuf.at[b],
                                      sem_g[b]),
                pltpu.make_async_copy(px_hbm.at[idx_ref], nbx.at[b],
                                      sem_g[b]),
                pltpu.make_async_copy(py_hbm.at[idx_ref], nby.at[b],
                                      sem_g[b]),
                pltpu.make_async_copy(pz_hbm.at[idx_ref], nbz.at[b],
                                      sem_g[b]),
            )

        def out_copy(b, c):
            return pltpu.make_async_copy(
                outbuf.at[b],
                out_hbm.at[pl.ds(c * CHUNK_PTS, CHUNK_PTS)], sem_o[b])

        def compute_chunk(b, c, j):
            cxv = lax.bitcast_convert_type(
                meta_v[b, pl.ds(CHUNK_EDGES, 16)], jnp.float32)
            cyv = lax.bitcast_convert_type(
                meta_v[b, pl.ds(CHUNK_EDGES + 16, 16)], jnp.float32)
            czv = lax.bitcast_convert_type(
                meta_v[b, pl.ds(CHUNK_EDGES + 32, 16)], jnp.float32)

            @pl.when(j >= 2)
            def _():
                out_copy(b, c).wait()

            for p in range(CHUNK_PTS):
                nx = nbx[b, pl.ds(p * KNN, KNN)]
                ny = nby[b, pl.ds(p * KNN, KNN)]
                nz = nbz[b, pl.ds(p * KNN, KNN)]
                dx = cxv[p] - nx
                dy = cyv[p] - ny
                dz = czv[p] - nz
                s = dx * dx + dy * dy + dz * dz
                bits = lax.bitcast_convert_type(s, jnp.int32)
                seed = jnp.int32(0x5F3759DF) - (bits >> 1)
                r = lax.bitcast_convert_type(seed, jnp.float32)
                hs = s * jnp.float32(-0.5)
                r = r * (hs * r * r + jnp.float32(1.5))
                r = r * (hs * r * r + jnp.float32(1.5))
                r = r * (hs * r * r + jnp.float32(1.5))
                nr = s * r
                acc_s = [jnp.zeros((16,), jnp.float32) for _ in range(4)]
                acc_a = [jnp.zeros((16,), jnp.float32) for _ in range(4)]
                acc_f = [jnp.zeros((16,), jnp.float32) for _ in range(4)]
                base = p * KNN
                for k in range(KNN):
                    dxk = dx[k]
                    dyk = dy[k]
                    dzk = dz[k]
                    nrk = nr[k]
                    for v in range(4):
                        z = (dxk * wvec[v][0] + dyk * wvec[v][1]
                             + dzk * wvec[v][2] + nrk * wvec[v][3]
                             + bvec[v])
                        acc_s[v] = acc_s[v] + z
                        acc_a[v] = acc_a[v] + jnp.abs(z)
                        acc_f[v] = acc_f[v] + featbuf[b, base + k,
                                                      pl.ds(16 * v, 16)]
                for v in range(4):
                    outbuf[b, p, pl.ds(16 * v, 16)] = (
                        acc_s[v] * jnp.float32(0.6 / KNN)
                        + acc_a[v] * jnp.float32(0.4 / KNN))
                    outbuf[b, p, pl.ds(64 + 16 * v, 16)] = (
                        acc_f[v] * jnp.float32(1.0 / KNN))
            out_copy(b, c).start()

        c0 = wid
        meta_copy(0, c0).start()

        @pl.when(c0 + NWORKERS < NCHUNKS)
        def _():
            meta_copy(1, c0 + NWORKERS).start()
        meta_copy(0, c0).wait()
        for cp in gather_copies(0):
            cp.start()

        def outer_body(o, carry):
            for bb in range(2):
                j = 2 * o + bb
                c = wid + NWORKERS * j
                c_nxt = c + NWORKERS
                c_nx2 = c_nxt + NWORKERS
                bo = 1 - bb

                @pl.when(c_nxt < NCHUNKS)
                def _(bo=bo, c_nxt=c_nxt):
                    meta_copy(bo, c_nxt).wait()
                    for cp in gather_copies(bo):
                        cp.start()

                @pl.when(c < NCHUNKS)
                def _(bb=bb, c=c, c_nx2=c_nx2, j=j):
                    for cp in gather_copies(bb):
                        cp.wait()

                    @pl.when(c_nx2 < NCHUNKS)
                    def _():
                        meta_copy(bb, c_nx2).start()
                    compute_chunk(bb, c, j)
            return carry

        lax.fori_loop(0, VITERS // 2, outer_body, 0)
        out_copy(0, wid).wait()
        out_copy(1, wid).wait()

    return lfa_kernel


_SC_CALL = _build_sc_call()


def kernel(points, features, knn_indices, W, b):
    B, N, D = points.shape
    pts = points.reshape(B * N, D).astype(jnp.float32)
    feat_flat = features.reshape(B * N, features.shape[-1]).astype(jnp.float32)
    px = pts[:, 0]
    py = pts[:, 1]
    pz = pts[:, 2]
    offs = (jnp.arange(B, dtype=jnp.int32) * N)[:, None, None]
    idx_flat = (knn_indices.astype(jnp.int32) + offs).reshape(NCHUNKS,
                                                              CHUNK_EDGES)
    zpad = jnp.zeros((NCHUNKS, 8), jnp.int32)

    def cfield(v):
        return lax.bitcast_convert_type(v.reshape(NCHUNKS, CHUNK_PTS),
                                        jnp.int32)

    meta = jnp.concatenate(
        [idx_flat, cfield(px), zpad, cfield(py), zpad, cfield(pz), zpad],
        axis=1).reshape(-1)
    wt = W.astype(jnp.float32).T
    out = _SC_CALL(meta, feat_flat, px, py, pz, wt, b.astype(jnp.float32))
    return out.reshape(B, N, _OUT_D)
